# baseline (speedup 1.0000x reference)
import functools

import jax
import jax.numpy as jnp
from jax.experimental import pallas as pl
from jax.experimental.pallas import tpu as pltpu

_BLOCK_W = 8192

_KS0 = 0
_KS1 = 42
_KS2 = (0x1BD11BDA ^ _KS0 ^ _KS1) & 0xFFFFFFFF
_ROTATIONS = ((13, 15, 26, 6), (17, 29, 16, 24))


def _threefry_bits(flat_idx):
    ks = (jnp.uint32(_KS0), jnp.uint32(_KS1), jnp.uint32(_KS2))
    x1 = flat_idx + ks[1]
    x0 = x1
    x1 = x0 ^ ((x1 << 13) | (x1 >> 19))
    first = True
    for i in range(5):
        for r in _ROTATIONS[i % 2]:
            if first:
                first = False
                continue
            x0 = x0 + x1
            x1 = (x1 << r) | (x1 >> (32 - r))
            x1 = x0 ^ x1
        x0 = x0 + ks[(i + 1) % 3]
        x1 = x1 + ks[(i + 2) % 3] + jnp.uint32(i + 1)
    return x0 ^ x1


def _gumbel(flat_idx):
    bits = _threefry_bits(flat_idx)
    fbits = (bits >> 9) | jnp.uint32(0x3F800000)
    u = jax.lax.bitcast_convert_type(fbits, jnp.float32) - 1.0
    return -jnp.log(-jnp.log(u + 1e-20) + 1e-20)


_CHUNK_W = 512


def _partial_argmax_kernel(x_ref, m_ref, a_ref, *, ncols, block_w):
    j = pl.program_id(0)
    rows = x_ref.shape[0]
    cshape = (rows, _CHUNK_W)
    col0 = (jax.lax.broadcasted_iota(jnp.int32, cshape, 1)
            + j * block_w)
    row = jax.lax.broadcasted_iota(jnp.int32, cshape, 0)
    flat0 = (row * ncols + col0).astype(jnp.uint32)
    rowlim = ((row[:, :1] + 1) * ncols).astype(jnp.uint32)
    bm = None
    for k in range(block_w // _CHUNK_W):
        x = x_ref[:, k * _CHUNK_W:(k + 1) * _CHUNK_W]
        flat = flat0 + jnp.uint32(k * _CHUNK_W)
        z = x + _gumbel(flat)
        z = jnp.where(flat < rowlim, z, -jnp.inf)
        m = jnp.max(z, axis=1, keepdims=True)
        iflat = jax.lax.bitcast_convert_type(flat, jnp.int32)
        cand = jnp.where(z == m, iflat, jnp.int32(2**31 - 1))
        a = jnp.min(cand, axis=1, keepdims=True)
        if bm is None:
            bm, ba = m, a
        else:
            upd = m > bm
            bm = jnp.where(upd, m, bm)
            ba = jnp.where(upd, a, ba)
    m_ref[0] = bm
    a_ref[0] = ba


def _merge_argmax(m_all, a_all, ncols):
    gmax = jnp.max(m_all, axis=0)
    cand = jnp.where(m_all == gmax[None], a_all, jnp.int32(2**31 - 1))
    fmin = jnp.min(cand, axis=0)
    row = jax.lax.broadcasted_iota(jnp.int32, fmin.shape, 0)
    return fmin - row * ncols


_OUT_W = 16384


def _onehot_kernel(m_ref, a_ref, o_ref, *, ncols):
    j = pl.program_id(0)
    idx = _merge_argmax(m_ref[...], a_ref[...], ncols)
    col = (jax.lax.broadcasted_iota(jnp.int32, o_ref.shape, 1)
           + j * _OUT_W)
    o_ref[...] = (col == idx).astype(jnp.float32)


@jax.jit
def kernel(logits):
    rows, ncols = logits.shape
    nb = pl.cdiv(ncols, _BLOCK_W)

    m_all, a_all = pl.pallas_call(
        functools.partial(_partial_argmax_kernel, ncols=ncols,
                          block_w=_BLOCK_W),
        grid=(nb,),
        in_specs=[pl.BlockSpec((rows, _BLOCK_W), lambda j: (0, j))],
        out_specs=[
            pl.BlockSpec((1, rows, 1), lambda j: (j, 0, 0)),
            pl.BlockSpec((1, rows, 1), lambda j: (j, 0, 0)),
        ],
        out_shape=[
            jax.ShapeDtypeStruct((nb, rows, 1), jnp.float32),
            jax.ShapeDtypeStruct((nb, rows, 1), jnp.int32),
        ],
        compiler_params=pltpu.CompilerParams(
            dimension_semantics=("parallel",)),
    )(logits)

    out = pl.pallas_call(
        functools.partial(_onehot_kernel, ncols=ncols),
        grid=(pl.cdiv(ncols, _OUT_W),),
        in_specs=[
            pl.BlockSpec((nb, rows, 1), lambda j: (0, 0, 0)),
            pl.BlockSpec((nb, rows, 1), lambda j: (0, 0, 0)),
        ],
        out_specs=pl.BlockSpec((rows, _OUT_W), lambda j: (0, j)),
        out_shape=jax.ShapeDtypeStruct((rows, ncols), jnp.float32),
        compiler_params=pltpu.CompilerParams(
            dimension_semantics=("parallel",)),
    )(m_all, a_all)
    return out


# speedup vs baseline: 1.0247x; 1.0247x over previous
import functools

import jax
import jax.numpy as jnp
from jax.experimental import pallas as pl
from jax.experimental.pallas import tpu as pltpu

_BLOCK_W = 2048

_KS0 = 0
_KS1 = 42
_KS2 = (0x1BD11BDA ^ _KS0 ^ _KS1) & 0xFFFFFFFF
_ROTATIONS = ((13, 15, 26, 6), (17, 29, 16, 24))


def _threefry_bits(flat_idx):
    ks = (jnp.uint32(_KS0), jnp.uint32(_KS1), jnp.uint32(_KS2))
    x1 = flat_idx + ks[1]
    x0 = x1
    x1 = x0 ^ ((x1 << 13) | (x1 >> 19))
    first = True
    for i in range(5):
        for r in _ROTATIONS[i % 2]:
            if first:
                first = False
                continue
            x0 = x0 + x1
            x1 = (x1 << r) | (x1 >> (32 - r))
            x1 = x0 ^ x1
        x0 = x0 + ks[(i + 1) % 3]
        x1 = x1 + ks[(i + 2) % 3] + jnp.uint32(i + 1)
    return x0 ^ x1


def _gumbel(flat_idx):
    bits = _threefry_bits(flat_idx)
    fbits = (bits >> 9) | jnp.uint32(0x3F800000)
    u = jax.lax.bitcast_convert_type(fbits, jnp.float32) - 1.0
    return -jnp.log(-jnp.log(u + 1e-20) + 1e-20)


_CHUNK_W = 512


def _partial_argmax_kernel(x_ref, m_ref, a_ref, *, ncols, block_w):
    j = pl.program_id(0)
    rows = x_ref.shape[0]
    cshape = (rows, _CHUNK_W)
    col0 = (jax.lax.broadcasted_iota(jnp.int32, cshape, 1)
            + j * block_w)
    row = jax.lax.broadcasted_iota(jnp.int32, cshape, 0)
    flat0 = (row * ncols + col0).astype(jnp.uint32)
    rowlim = ((row[:, :1] + 1) * ncols).astype(jnp.uint32)
    bm = None
    for k in range(block_w // _CHUNK_W):
        x = x_ref[:, k * _CHUNK_W:(k + 1) * _CHUNK_W]
        flat = flat0 + jnp.uint32(k * _CHUNK_W)
        z = x + _gumbel(flat)
        z = jnp.where(flat < rowlim, z, -jnp.inf)
        m = jnp.max(z, axis=1, keepdims=True)
        iflat = jax.lax.bitcast_convert_type(flat, jnp.int32)
        cand = jnp.where(z == m, iflat, jnp.int32(2**31 - 1))
        a = jnp.min(cand, axis=1, keepdims=True)
        if bm is None:
            bm, ba = m, a
        else:
            upd = m > bm
            bm = jnp.where(upd, m, bm)
            ba = jnp.where(upd, a, ba)
    m_ref[0] = bm
    a_ref[0] = ba


def _merge_argmax(m_all, a_all, ncols):
    gmax = jnp.max(m_all, axis=0)
    cand = jnp.where(m_all == gmax[None], a_all, jnp.int32(2**31 - 1))
    fmin = jnp.min(cand, axis=0)
    row = jax.lax.broadcasted_iota(jnp.int32, fmin.shape, 0)
    return fmin - row * ncols


_OUT_W = 16384


def _onehot_kernel(m_ref, a_ref, o_ref, *, ncols):
    j = pl.program_id(0)
    idx = _merge_argmax(m_ref[...], a_ref[...], ncols)
    col = (jax.lax.broadcasted_iota(jnp.int32, o_ref.shape, 1)
           + j * _OUT_W)
    o_ref[...] = (col == idx).astype(jnp.float32)


@jax.jit
def kernel(logits):
    rows, ncols = logits.shape
    nb = pl.cdiv(ncols, _BLOCK_W)

    m_all, a_all = pl.pallas_call(
        functools.partial(_partial_argmax_kernel, ncols=ncols,
                          block_w=_BLOCK_W),
        grid=(nb,),
        in_specs=[pl.BlockSpec((rows, _BLOCK_W), lambda j: (0, j))],
        out_specs=[
            pl.BlockSpec((1, rows, 1), lambda j: (j, 0, 0)),
            pl.BlockSpec((1, rows, 1), lambda j: (j, 0, 0)),
        ],
        out_shape=[
            jax.ShapeDtypeStruct((nb, rows, 1), jnp.float32),
            jax.ShapeDtypeStruct((nb, rows, 1), jnp.int32),
        ],
        compiler_params=pltpu.CompilerParams(
            dimension_semantics=("parallel",)),
    )(logits)

    out = pl.pallas_call(
        functools.partial(_onehot_kernel, ncols=ncols),
        grid=(pl.cdiv(ncols, _OUT_W),),
        in_specs=[
            pl.BlockSpec((nb, rows, 1), lambda j: (0, 0, 0)),
            pl.BlockSpec((nb, rows, 1), lambda j: (0, 0, 0)),
        ],
        out_specs=pl.BlockSpec((rows, _OUT_W), lambda j: (0, j)),
        out_shape=jax.ShapeDtypeStruct((rows, ncols), jnp.float32),
        compiler_params=pltpu.CompilerParams(
            dimension_semantics=("parallel",)),
    )(m_all, a_all)
    return out


# speedup vs baseline: 1.0306x; 1.0058x over previous
import functools

import jax
import jax.numpy as jnp
from jax.experimental import pallas as pl
from jax.experimental.pallas import tpu as pltpu

_BLOCK_W = 4096

_KS0 = 0
_KS1 = 42
_KS2 = (0x1BD11BDA ^ _KS0 ^ _KS1) & 0xFFFFFFFF
_ROTATIONS = ((13, 15, 26, 6), (17, 29, 16, 24))


def _threefry_bits(flat_idx):
    ks = (jnp.uint32(_KS0), jnp.uint32(_KS1), jnp.uint32(_KS2))
    x0 = jnp.zeros_like(flat_idx) + ks[0]
    x1 = flat_idx + ks[1]
    for i in range(5):
        for r in _ROTATIONS[i % 2]:
            x0 = x0 + x1
            x1 = (x1 << r) | (x1 >> (32 - r))
            x1 = x0 ^ x1
        x0 = x0 + ks[(i + 1) % 3]
        x1 = x1 + ks[(i + 2) % 3] + jnp.uint32(i + 1)
    return x0 ^ x1


def _gumbel(flat_idx):
    bits = _threefry_bits(flat_idx)
    fbits = (bits >> 9) | jnp.uint32(0x3F800000)
    u = jax.lax.bitcast_convert_type(fbits, jnp.float32) - 1.0
    return -jnp.log(-jnp.log(u + 1e-20) + 1e-20)


_CHUNK_W = 512


def _partial_argmax_kernel(x_ref, m_ref, a_ref, *, ncols, block_w):
    j = pl.program_id(0)
    rows = x_ref.shape[0]
    cshape = (rows, _CHUNK_W)
    col0 = (jax.lax.broadcasted_iota(jnp.int32, cshape, 1)
            + j * block_w)
    row = jax.lax.broadcasted_iota(jnp.int32, cshape, 0)
    flat0 = (row * ncols + col0).astype(jnp.uint32)
    bm = None
    for k in range(block_w // _CHUNK_W):
        x = x_ref[:, k * _CHUNK_W:(k + 1) * _CHUNK_W]
        col = col0 + (k * _CHUNK_W)
        flat = flat0 + jnp.uint32(k * _CHUNK_W)
        z = x + _gumbel(flat)
        z = jnp.where(col < ncols, z, -jnp.inf)
        m = jnp.max(z, axis=1, keepdims=True)
        cand = jnp.where(z == m, col, jnp.int32(2**31 - 1))
        a = jnp.min(cand, axis=1, keepdims=True)
        if bm is None:
            bm, ba = m, a
        else:
            upd = m > bm
            bm = jnp.where(upd, m, bm)
            ba = jnp.where(upd, a, ba)
    m_ref[0] = bm
    a_ref[0] = ba


def _merge_argmax(m_all, a_all):
    gmax = jnp.max(m_all, axis=0)
    cand = jnp.where(m_all == gmax[None], a_all, jnp.int32(2**31 - 1))
    return jnp.min(cand, axis=0)


_OUT_W = 16384


def _onehot_kernel(m_ref, a_ref, o_ref):
    j = pl.program_id(0)
    idx = _merge_argmax(m_ref[...], a_ref[...])
    col = (jax.lax.broadcasted_iota(jnp.int32, o_ref.shape, 1)
           + j * _OUT_W)
    o_ref[...] = (col == idx).astype(jnp.float32)


@jax.jit
def kernel(logits):
    rows, ncols = logits.shape
    nb = pl.cdiv(ncols, _BLOCK_W)

    m_all, a_all = pl.pallas_call(
        functools.partial(_partial_argmax_kernel, ncols=ncols,
                          block_w=_BLOCK_W),
        grid=(nb,),
        in_specs=[pl.BlockSpec((rows, _BLOCK_W), lambda j: (0, j))],
        out_specs=[
            pl.BlockSpec((1, rows, 1), lambda j: (j, 0, 0)),
            pl.BlockSpec((1, rows, 1), lambda j: (j, 0, 0)),
        ],
        out_shape=[
            jax.ShapeDtypeStruct((nb, rows, 1), jnp.float32),
            jax.ShapeDtypeStruct((nb, rows, 1), jnp.int32),
        ],
        compiler_params=pltpu.CompilerParams(
            dimension_semantics=("parallel",)),
    )(logits)

    out = pl.pallas_call(
        _onehot_kernel,
        grid=(pl.cdiv(ncols, _OUT_W),),
        in_specs=[
            pl.BlockSpec((nb, rows, 1), lambda j: (0, 0, 0)),
            pl.BlockSpec((nb, rows, 1), lambda j: (0, 0, 0)),
        ],
        out_specs=pl.BlockSpec((rows, _OUT_W), lambda j: (0, j)),
        out_shape=jax.ShapeDtypeStruct((rows, ncols), jnp.float32),
        compiler_params=pltpu.CompilerParams(
            dimension_semantics=("parallel",)),
    )(m_all, a_all)
    return out


# speedup vs baseline: 1.0343x; 1.0036x over previous
import functools

import jax
import jax.numpy as jnp
from jax.experimental import pallas as pl
from jax.experimental.pallas import tpu as pltpu

_BLOCK_W = 4096

_KS0 = 0
_KS1 = 42
_KS2 = (0x1BD11BDA ^ _KS0 ^ _KS1) & 0xFFFFFFFF
_ROTATIONS = ((13, 15, 26, 6), (17, 29, 16, 24))


def _threefry_bits(flat_idx):
    ks = (jnp.uint32(_KS0), jnp.uint32(_KS1), jnp.uint32(_KS2))
    x0 = jnp.zeros_like(flat_idx) + ks[0]
    x1 = flat_idx + ks[1]
    for i in range(5):
        for r in _ROTATIONS[i % 2]:
            x0 = x0 + x1
            x1 = (x1 << r) | (x1 >> (32 - r))
            x1 = x0 ^ x1
        x0 = x0 + ks[(i + 1) % 3]
        x1 = x1 + ks[(i + 2) % 3] + jnp.uint32(i + 1)
    return x0 ^ x1


def _gumbel(flat_idx):
    bits = _threefry_bits(flat_idx)
    fbits = (bits >> 9) | jnp.uint32(0x3F800000)
    u = jax.lax.bitcast_convert_type(fbits, jnp.float32) - 1.0
    return -jnp.log(-jnp.log(u + 1e-20) + 1e-20)


_CHUNK_W = 1024


def _partial_argmax_kernel(x_ref, m_ref, a_ref, *, ncols, block_w):
    j = pl.program_id(0)
    rows = x_ref.shape[0]
    cshape = (rows, _CHUNK_W)
    col0 = (jax.lax.broadcasted_iota(jnp.int32, cshape, 1)
            + j * block_w)
    row = jax.lax.broadcasted_iota(jnp.int32, cshape, 0)
    flat0 = (row * ncols + col0).astype(jnp.uint32)
    bm = None
    for k in range(block_w // _CHUNK_W):
        x = x_ref[:, k * _CHUNK_W:(k + 1) * _CHUNK_W]
        col = col0 + (k * _CHUNK_W)
        flat = flat0 + jnp.uint32(k * _CHUNK_W)
        z = x + _gumbel(flat)
        z = jnp.where(col < ncols, z, -jnp.inf)
        m = jnp.max(z, axis=1, keepdims=True)
        cand = jnp.where(z == m, col, jnp.int32(2**31 - 1))
        a = jnp.min(cand, axis=1, keepdims=True)
        if bm is None:
            bm, ba = m, a
        else:
            upd = m > bm
            bm = jnp.where(upd, m, bm)
            ba = jnp.where(upd, a, ba)
    m_ref[0] = bm
    a_ref[0] = ba


def _merge_argmax(m_all, a_all):
    gmax = jnp.max(m_all, axis=0)
    cand = jnp.where(m_all == gmax[None], a_all, jnp.int32(2**31 - 1))
    return jnp.min(cand, axis=0)


_OUT_W = 16384


def _onehot_kernel(m_ref, a_ref, o_ref):
    j = pl.program_id(0)
    idx = _merge_argmax(m_ref[...], a_ref[...])
    col = (jax.lax.broadcasted_iota(jnp.int32, o_ref.shape, 1)
           + j * _OUT_W)
    o_ref[...] = (col == idx).astype(jnp.float32)


@jax.jit
def kernel(logits):
    rows, ncols = logits.shape
    nb = pl.cdiv(ncols, _BLOCK_W)

    m_all, a_all = pl.pallas_call(
        functools.partial(_partial_argmax_kernel, ncols=ncols,
                          block_w=_BLOCK_W),
        grid=(nb,),
        in_specs=[pl.BlockSpec((rows, _BLOCK_W), lambda j: (0, j))],
        out_specs=[
            pl.BlockSpec((1, rows, 1), lambda j: (j, 0, 0)),
            pl.BlockSpec((1, rows, 1), lambda j: (j, 0, 0)),
        ],
        out_shape=[
            jax.ShapeDtypeStruct((nb, rows, 1), jnp.float32),
            jax.ShapeDtypeStruct((nb, rows, 1), jnp.int32),
        ],
        compiler_params=pltpu.CompilerParams(
            dimension_semantics=("parallel",)),
    )(logits)

    out = pl.pallas_call(
        _onehot_kernel,
        grid=(pl.cdiv(ncols, _OUT_W),),
        in_specs=[
            pl.BlockSpec((nb, rows, 1), lambda j: (0, 0, 0)),
            pl.BlockSpec((nb, rows, 1), lambda j: (0, 0, 0)),
        ],
        out_specs=pl.BlockSpec((rows, _OUT_W), lambda j: (0, j)),
        out_shape=jax.ShapeDtypeStruct((rows, ncols), jnp.float32),
        compiler_params=pltpu.CompilerParams(
            dimension_semantics=("parallel",)),
    )(m_all, a_all)
    return out
